# baseline (speedup 1.0000x reference)
import jax
import jax.numpy as jnp
from jax import lax
from jax.experimental import pallas as pl
from jax.experimental.pallas import tpu as pltpu
from jax.experimental.pallas import tpu_sc as plsc

N_NODES = 10000
N_EDGES = 320000
D_FEAT = 128

NC = 2
NS = 16

DHALF = D_FEAT // NC
E_PER_TILE = N_EDGES // NS
BATCH = 100
NBATCH = E_PER_TILE // BATCH
HALF = 3
RING = 2 * HALF
NITER = NBATCH // RING
NLEFT = NBATCH - NITER * RING
N_PAD = 10240
ROWS_PER_TILE = N_PAD // NS
LAST_ROWS = N_NODES - 15 * ROWS_PER_TILE


def _sc_kernel(x2, src4, dst3, out,
               acc, idx_s, idx_d, rows, gsem, ssem):
  c = lax.axis_index("c")
  s = lax.axis_index("s")
  r0 = s * ROWS_PER_TILE

  a = pltpu.async_copy(src4.at[c, s], idx_s, gsem[0])
  b = pltpu.async_copy(dst3.at[s], idx_d, gsem[1])

  zv = jnp.zeros((16,), jnp.float32)
  def zstep(i, carry):
    for j in range(DHALF // 16):
      rows[0][i, pl.ds(j * 16, 16)] = zv
    return carry
  lax.fori_loop(0, BATCH, zstep, 0)
  ZC = 80
  for k in range(ROWS_PER_TILE // ZC):
    pltpu.sync_copy(rows[0].at[pl.ds(0, ZC)],
                    acc.at[pl.ds(r0 + k * ZC, ZC)])
  a.wait()
  b.wait()
  plsc.subcore_barrier()

  def body(xh):
    def issue_gather(b, u):
      return pltpu.async_copy(xh.at[idx_s.at[b]], rows[u], gsem[u])

    def wait_gather(b, u):
      pltpu.make_async_copy(xh.at[idx_s.at[b]], rows[u], gsem[u]).wait()

    def issue_scatter(b, u):
      return pltpu.async_copy(rows[u], acc.at[idx_d.at[b]], ssem[u],
                              add=True)

    def wait_scatter(b, u):
      pltpu.make_async_copy(rows[u], acc.at[idx_d.at[b]], ssem[u]).wait()

    for u in range(HALF):
      issue_gather(u, u)

    def step(q, carry):
      bA = q * RING
      bB = bA + HALF
      for u in range(HALF):
        issue_gather(bB + u, HALF + u)
      for u in range(HALF):
        wait_gather(bA + u, u)
        issue_scatter(bA + u, u)
      for u in range(HALF):
        wait_scatter(bA + u, u)

      @pl.when(q < NITER - 1)
      def _():
        for u in range(HALF):
          issue_gather(bA + RING + u, u)

      for u in range(HALF):
        wait_gather(bB + u, HALF + u)
        issue_scatter(bB + u, HALF + u)
      for u in range(HALF):
        wait_scatter(bB + u, HALF + u)
      return carry
    lax.fori_loop(0, NITER, step, 0)

    for u in range(NLEFT):
      b = NITER * RING + u
      issue_gather(b, u).wait()
      issue_scatter(b, u).wait()

  body(x2)

  plsc.subcore_barrier()

  @pl.when(s < NS - 1)
  def _():
    pltpu.sync_copy(
        acc.at[pl.ds(r0, ROWS_PER_TILE)],
        out.at[pl.ds(r0, ROWS_PER_TILE), pl.ds(c * DHALF, DHALF)],
    )

  @pl.when(s == NS - 1)
  def _():
    pltpu.sync_copy(
        acc.at[pl.ds(r0, LAST_ROWS)],
        out.at[pl.ds(r0, LAST_ROWS), pl.ds(c * DHALF, DHALF)],
    )


@jax.jit
def _path_add(x2, src4, dst3):
  mesh = plsc.VectorSubcoreMesh(core_axis_name="c", subcore_axis_name="s")
  return pl.kernel(
      _sc_kernel,
      out_type=jax.ShapeDtypeStruct((N_NODES, D_FEAT), jnp.float32),
      mesh=mesh,
      scratch_types=[
          pltpu.VMEM_SHARED((N_PAD, DHALF), jnp.float32),
          pltpu.VMEM((NBATCH, BATCH), jnp.int32),
          pltpu.VMEM((NBATCH, BATCH), jnp.int32),
          [pltpu.VMEM((BATCH, DHALF), jnp.float32)
           for _ in range(RING)],
          [pltpu.SemaphoreType.DMA for _ in range(RING)],
          [pltpu.SemaphoreType.DMA for _ in range(RING)],
      ],
      compiler_params=pltpu.CompilerParams(use_tc_tiling_on_sc=False),
      name="path_add_sc",
  )(x2, src4, dst3)


def kernel(x, edge_index):
  x2 = x.reshape(NC * N_NODES, DHALF)
  src2 = edge_index[0] * 2
  src4 = jnp.stack([src2, src2 + 1]).reshape(NC, NS, NBATCH, BATCH)
  dst3 = edge_index[1].reshape(NS, NBATCH, BATCH)
  return _path_add(x2, src4, dst3)


# speedup vs baseline: 1.0428x; 1.0428x over previous
import jax
import jax.numpy as jnp
from jax import lax
from jax.experimental import pallas as pl
from jax.experimental.pallas import tpu as pltpu
from jax.experimental.pallas import tpu_sc as plsc

N_NODES = 10000
N_EDGES = 320000
D_FEAT = 128

NC = 2
NS = 16

DHALF = D_FEAT // NC
E_PER_TILE = N_EDGES // NS
BATCH = 160
NBATCH = E_PER_TILE // BATCH
HALF = 2
RING = 2 * HALF
NITER = NBATCH // RING
NLEFT = NBATCH - NITER * RING
N_PAD = 10240
ROWS_PER_TILE = N_PAD // NS
LAST_ROWS = N_NODES - 15 * ROWS_PER_TILE


def _sc_kernel(x2, src4, dst3, out,
               acc, idx_s, idx_d, rows, gsem, ssem):
  c = lax.axis_index("c")
  s = lax.axis_index("s")
  r0 = s * ROWS_PER_TILE

  a = pltpu.async_copy(src4.at[c, s], idx_s, gsem[0])
  b = pltpu.async_copy(dst3.at[s], idx_d, gsem[1])

  zv = jnp.zeros((16,), jnp.float32)
  def zstep(i, carry):
    for j in range(DHALF // 16):
      rows[RING - 1][i, pl.ds(j * 16, 16)] = zv
    return carry
  lax.fori_loop(0, BATCH, zstep, 0)
  a.wait()

  for u in range(HALF):
    pltpu.async_copy(x2.at[idx_s.at[u]], rows[u], gsem[u])

  for k in range(ROWS_PER_TILE // BATCH):
    pltpu.sync_copy(rows[RING - 1], acc.at[pl.ds(r0 + k * BATCH, BATCH)])
  b.wait()
  plsc.subcore_barrier()

  def body(xh):
    def issue_gather(b, u):
      return pltpu.async_copy(xh.at[idx_s.at[b]], rows[u], gsem[u])

    def wait_gather(b, u):
      pltpu.make_async_copy(xh.at[idx_s.at[b]], rows[u], gsem[u]).wait()

    def issue_scatter(b, u):
      return pltpu.async_copy(rows[u], acc.at[idx_d.at[b]], ssem[u],
                              add=True)

    def wait_scatter(b, u):
      pltpu.make_async_copy(rows[u], acc.at[idx_d.at[b]], ssem[u]).wait()

    def step(q, carry):
      bA = q * RING
      bB = bA + HALF
      for u in range(HALF):
        issue_gather(bB + u, HALF + u)
      for u in range(HALF):
        wait_gather(bA + u, u)
        issue_scatter(bA + u, u)
      for u in range(HALF):
        wait_scatter(bA + u, u)

      @pl.when(q < NITER - 1)
      def _():
        for u in range(HALF):
          issue_gather(bA + RING + u, u)

      for u in range(HALF):
        wait_gather(bB + u, HALF + u)
        issue_scatter(bB + u, HALF + u)
      for u in range(HALF):
        wait_scatter(bB + u, HALF + u)
      return carry
    lax.fori_loop(0, NITER, step, 0)

    for u in range(NLEFT):
      b = NITER * RING + u
      issue_gather(b, u).wait()
      issue_scatter(b, u).wait()

  body(x2)

  plsc.subcore_barrier()

  @pl.when(s < NS - 1)
  def _():
    pltpu.sync_copy(
        acc.at[pl.ds(r0, ROWS_PER_TILE)],
        out.at[pl.ds(r0, ROWS_PER_TILE), pl.ds(c * DHALF, DHALF)],
    )

  @pl.when(s == NS - 1)
  def _():
    pltpu.sync_copy(
        acc.at[pl.ds(r0, LAST_ROWS)],
        out.at[pl.ds(r0, LAST_ROWS), pl.ds(c * DHALF, DHALF)],
    )


@jax.jit
def _path_add(x2, src4, dst3):
  mesh = plsc.VectorSubcoreMesh(core_axis_name="c", subcore_axis_name="s")
  return pl.kernel(
      _sc_kernel,
      out_type=jax.ShapeDtypeStruct((N_NODES, D_FEAT), jnp.float32),
      mesh=mesh,
      scratch_types=[
          pltpu.VMEM_SHARED((N_PAD, DHALF), jnp.float32),
          pltpu.VMEM((NBATCH, BATCH), jnp.int32),
          pltpu.VMEM((NBATCH, BATCH), jnp.int32),
          [pltpu.VMEM((BATCH, DHALF), jnp.float32)
           for _ in range(RING)],
          [pltpu.SemaphoreType.DMA for _ in range(RING)],
          [pltpu.SemaphoreType.DMA for _ in range(RING)],
      ],
      compiler_params=pltpu.CompilerParams(use_tc_tiling_on_sc=False),
      name="path_add_sc",
  )(x2, src4, dst3)


def kernel(x, edge_index):
  x2 = x.reshape(NC * N_NODES, DHALF)
  src2 = edge_index[0] * 2
  src4 = jnp.stack([src2, src2 + 1]).reshape(NC, NS, NBATCH, BATCH)
  dst3 = edge_index[1].reshape(NS, NBATCH, BATCH)
  return _path_add(x2, src4, dst3)
